# baseline (speedup 1.0000x reference)
import functools

import jax
import jax.numpy as jnp
from jax import lax
from jax.experimental import pallas as pl
from jax.experimental.pallas import tpu as pltpu
from jax.experimental.pallas import tpu_sc as plsc


def _sc_gather_pair(h, src, dst):
    N, D = h.shape
    E = src.shape[0]
    NW = 32
    per_w = E // NW
    K = 200
    nch = per_w // K
    mesh = plsc.VectorSubcoreMesh(core_axis_name="c", subcore_axis_name="s")

    @functools.partial(
        pl.kernel, mesh=mesh,
        out_type=[jax.ShapeDtypeStruct((E, D), jnp.float32),
                  jax.ShapeDtypeStruct((E, D), jnp.float32)],
        scratch_types=[pltpu.VMEM((K,), jnp.int32),
                       pltpu.VMEM((K, D), jnp.float32),
                       pltpu.SemaphoreType.DMA],
    )
    def gk(h_hbm, src_hbm, dst_hbm, hs_hbm, hd_hbm, idx_v, rows_v, sem):
        wid = lax.axis_index("s") * 2 + lax.axis_index("c")
        base = wid * per_w
        for j in range(nch):
            off = base + j * K
            pltpu.sync_copy(src_hbm.at[pl.ds(off, K)], idx_v)
            pltpu.async_copy(h_hbm.at[idx_v], rows_v, sem).wait()
            pltpu.sync_copy(rows_v, hs_hbm.at[pl.ds(off, K), :])
            pltpu.sync_copy(dst_hbm.at[pl.ds(off, K)], idx_v)
            pltpu.async_copy(h_hbm.at[idx_v], rows_v, sem).wait()
            pltpu.sync_copy(rows_v, hd_hbm.at[pl.ds(off, K), :])

    return gk(h, src, dst)


def _edge_mlp_body(hs_ref, hd_ref, ed_ref,
                   Wab_ref, Wbb_ref, Web_ref, bf_ref,
                   W2_ref, b2_ref, wc2_ref,
                   We1_ref, be1_ref, We2_ref, be2_ref,
                   m_ref, cw_ref, *, H):
    f32 = jnp.float32
    bf16 = jnp.bfloat16
    d = ed_ref[...]
    eh = d * We1_ref[...] + be1_ref[...]
    eh = eh * jax.nn.sigmoid(eh)
    ea = jnp.dot(eh, We2_ref[...], preferred_element_type=f32) + be2_ref[...]
    pre = (jnp.dot(hs_ref[...].astype(bf16), Wab_ref[...], preferred_element_type=f32)
           + jnp.dot(hd_ref[...].astype(bf16), Wbb_ref[...], preferred_element_type=f32)
           + jnp.dot(ea.astype(bf16), Web_ref[...], preferred_element_type=f32)
           + bf_ref[...])
    preb = pre.astype(bf16)
    act = preb * jax.nn.sigmoid(preb)
    a_node = act[:, :H]
    a_coord = act[:, H:]
    m_ref[...] = jnp.dot(a_node, W2_ref[...], preferred_element_type=f32) + b2_ref[...]
    cw_ref[...] = jnp.sum(a_coord * wc2_ref[...], axis=1)


def _sc_scatter(h, x0, x1, x2, xz, m, cw, src, dst):
    N, D = h.shape
    E = src.shape[0]
    HC = D // 2
    K = 200
    nh = E // 16 // K
    KX = 320
    ncx = E // 2 // KX
    rows_t = (N // 16) & ~7
    tail0 = rows_t * 16
    ntail = N - tail0
    mesh = plsc.VectorSubcoreMesh(core_axis_name="c", subcore_axis_name="s")

    @functools.partial(
        pl.kernel, mesh=mesh,
        out_type=[jax.ShapeDtypeStruct((N, D), jnp.float32)]
                 + [jax.ShapeDtypeStruct((N,), jnp.float32)] * 6,
        scratch_types=[pltpu.VMEM_SHARED((N, HC), jnp.float32),
                       pltpu.VMEM_SHARED((N,), jnp.float32),
                       pltpu.VMEM_SHARED((N,), jnp.float32),
                       pltpu.VMEM_SHARED((N,), jnp.float32),
                       pltpu.VMEM((K,), jnp.int32),
                       pltpu.VMEM((K, HC), jnp.float32),
                       pltpu.VMEM((KX,), jnp.int32),
                       pltpu.VMEM((KX,), jnp.int32),
                       pltpu.VMEM((KX,), jnp.float32),
                       pltpu.VMEM((KX,), jnp.float32),
                       pltpu.VMEM((KX,), jnp.float32),
                       pltpu.VMEM((KX,), jnp.float32),
                       pltpu.SemaphoreType.DMA],
    )
    def sk(h_hbm, x0_hbm, x1_hbm, x2_hbm, xz_hbm, m_hbm, cw_hbm,
           src_hbm, dst_hbm, ho_hbm, p00, p01, p02, p10, p11, p12,
           hacc, xa0, xa1, xa2, dstv, mv, srcv, dstxv, cwv, xsv, xdv, cuv,
           sem):
        c = lax.axis_index("c")
        s = lax.axis_index("s")
        r0 = pl.multiple_of(s * rows_t, 8)
        col = pl.multiple_of(c * HC, HC)
        xaccs = [xa0, xa1, xa2]
        xcols = [x0_hbm, x1_hbm, x2_hbm]
        pltpu.sync_copy(h_hbm.at[pl.ds(r0, rows_t), pl.ds(col, HC)],
                        hacc.at[pl.ds(r0, rows_t), :])
        if ntail:
            @pl.when(s == 0)
            def _():
                pltpu.sync_copy(h_hbm.at[pl.ds(tail0, ntail), pl.ds(col, HC)],
                                hacc.at[pl.ds(tail0, ntail), :])
        @pl.when(s == 0)
        def _():
            for q in range(3):
                pltpu.sync_copy(xz_hbm, xaccs[q])
        plsc.subcore_barrier()

        hbase = s * (E // 16)
        def hchunk(j, _):
            off = pl.multiple_of(hbase + j * K, 8)
            pltpu.sync_copy(dst_hbm.at[pl.ds(off, K)], dstv)
            pltpu.sync_copy(m_hbm.at[pl.ds(off, K), pl.ds(col, HC)], mv)
            pltpu.sync_copy(mv, hacc.at[dstv], add=True)
            return _
        lax.fori_loop(0, nh, hchunk, 0)

        nj = (ncx - s + 15) // 16
        def xchunk(j, _):
            cidx = s + j * 16
            off = pl.multiple_of(c * (E // 2) + cidx * KX, 8)
            pltpu.sync_copy(src_hbm.at[pl.ds(off, KX)], srcv)
            pltpu.sync_copy(dst_hbm.at[pl.ds(off, KX)], dstxv)
            pltpu.sync_copy(cw_hbm.at[pl.ds(off, KX)], cwv)
            for q in range(3):
                pltpu.async_copy(xcols[q].at[srcv], xsv, sem).wait()
                pltpu.async_copy(xcols[q].at[dstxv], xdv, sem).wait()
                for g in range(KX // 16):
                    d16 = pl.ds(g * 16, 16)
                    cuv[d16] = cwv[d16] * (xsv[d16] - xdv[d16])
                pltpu.sync_copy(cuv, xaccs[q].at[dstxv], add=True)
            return _
        lax.fori_loop(0, nj, xchunk, 0)
        plsc.subcore_barrier()

        pltpu.sync_copy(hacc.at[pl.ds(r0, rows_t), :],
                        ho_hbm.at[pl.ds(r0, rows_t), pl.ds(col, HC)])
        if ntail:
            @pl.when(s == 0)
            def _():
                pltpu.sync_copy(hacc.at[pl.ds(tail0, ntail), :],
                                ho_hbm.at[pl.ds(tail0, ntail), pl.ds(col, HC)])
        @pl.when((s == 0) & (c == 0))
        def _():
            for q, dst_ref in enumerate([p00, p01, p02]):
                pltpu.sync_copy(xaccs[q], dst_ref)
        @pl.when((s == 0) & (c == 1))
        def _():
            for q, dst_ref in enumerate([p10, p11, p12]):
                pltpu.sync_copy(xaccs[q], dst_ref)

    return sk(h, x0, x1, x2, xz, m, cw, src, dst)


def _pick_block(E):
    for b in (256, 128, 64, 32, 16, 8):
        if E % b == 0:
            return b
    return 8


def kernel(h, x, edge_index, edge_dist, W1, b1, W2, b2, Wc1, bc1, Wc2, We1, be1, We2, be2):
    N, D = h.shape
    E = edge_dist.shape[0]
    H = W1.shape[1]
    ED = We2.shape[0]
    B = _pick_block(E)
    src = edge_index[0]
    dst = edge_index[1]

    bf16 = jnp.bfloat16
    Wab = jnp.concatenate([W1[:D], Wc1[:D]], axis=1).astype(bf16)
    Wbb = jnp.concatenate([W1[D:2 * D], Wc1[D:2 * D]], axis=1).astype(bf16)
    Web = jnp.concatenate([W1[2 * D:], Wc1[2 * D:]], axis=1).astype(bf16)
    bf = jnp.concatenate([b1, bc1])[None, :]
    b2r = b2[None, :]
    wc2r = Wc2[:, 0][None, :]
    be1r = be1[None, :]
    be2r = be2[None, :]

    hs, hd = _sc_gather_pair(h, src, dst)
    ed2 = edge_dist[:, None]

    grid = (E // B,)
    full = lambda r, c: pl.BlockSpec((r, c), lambda i: (0, 0))
    m, cw = pl.pallas_call(
        functools.partial(_edge_mlp_body, H=H),
        grid=grid,
        in_specs=[
            pl.BlockSpec((B, D), lambda i: (i, 0)),
            pl.BlockSpec((B, D), lambda i: (i, 0)),
            pl.BlockSpec((B, 1), lambda i: (i, 0)),
            full(D, 2 * H), full(D, 2 * H), full(ED, 2 * H), full(1, 2 * H),
            full(H, D), full(1, D), full(1, H),
            full(1, ED), full(1, ED), full(ED, ED), full(1, ED),
        ],
        out_specs=[
            pl.BlockSpec((B, D), lambda i: (i, 0)),
            pl.BlockSpec((B,), lambda i: (i,)),
        ],
        out_shape=[
            jax.ShapeDtypeStruct((E, D), jnp.float32),
            jax.ShapeDtypeStruct((E,), jnp.float32),
        ],
    )(hs, hd, ed2, Wab, Wbb, Web, bf, W2.astype(bf16), b2r, wc2r, We1, be1r, We2, be2r)

    xz = jnp.zeros((N,), jnp.float32)
    h_out, p00, p01, p02, p10, p11, p12 = _sc_scatter(
        h, x[:, 0], x[:, 1], x[:, 2], xz, m, cw, src, dst)
    x_out = x + jnp.stack([p00 + p10, p01 + p11, p02 + p12], axis=1)
    return (h_out, x_out)


# speedup vs baseline: 1.3189x; 1.3189x over previous
import functools

import jax
import jax.numpy as jnp
from jax import lax
from jax.experimental import pallas as pl
from jax.experimental.pallas import tpu as pltpu
from jax.experimental.pallas import tpu_sc as plsc


def _sc_gather_pair(h, src, dst):
    N, D = h.shape
    E = src.shape[0]
    NW = 32
    per_w = E // NW
    K = 200
    nch = per_w // K
    mesh = plsc.VectorSubcoreMesh(core_axis_name="c", subcore_axis_name="s")

    @functools.partial(
        pl.kernel, mesh=mesh,
        out_type=[jax.ShapeDtypeStruct((E, D), jnp.float32),
                  jax.ShapeDtypeStruct((E, D), jnp.float32)],
        scratch_types=[pltpu.VMEM((K,), jnp.int32),
                       pltpu.VMEM((K, D), jnp.float32),
                       pltpu.SemaphoreType.DMA],
    )
    def gk(h_hbm, src_hbm, dst_hbm, hs_hbm, hd_hbm, idx_v, rows_v, sem):
        wid = lax.axis_index("s") * 2 + lax.axis_index("c")
        base = wid * per_w
        for j in range(nch):
            off = base + j * K
            pltpu.sync_copy(src_hbm.at[pl.ds(off, K)], idx_v)
            pltpu.async_copy(h_hbm.at[idx_v], rows_v, sem).wait()
            pltpu.sync_copy(rows_v, hs_hbm.at[pl.ds(off, K), :])
            pltpu.sync_copy(dst_hbm.at[pl.ds(off, K)], idx_v)
            pltpu.async_copy(h_hbm.at[idx_v], rows_v, sem).wait()
            pltpu.sync_copy(rows_v, hd_hbm.at[pl.ds(off, K), :])

    return gk(h, src, dst)


def _edge_mlp_body(hs_ref, hd_ref, ed_ref,
                   Wab_ref, Wbb_ref, Web_ref, bf_ref,
                   W2_ref, b2_ref, wc2_ref,
                   We1_ref, be1_ref, We2_ref, be2_ref,
                   m_ref, cw_ref, *, H):
    f32 = jnp.float32
    bf16 = jnp.bfloat16
    d = ed_ref[...]
    eh = d * We1_ref[...] + be1_ref[...]
    eh = eh * jax.nn.sigmoid(eh)
    ea = jnp.dot(eh, We2_ref[...], preferred_element_type=f32) + be2_ref[...]
    pre = (jnp.dot(hs_ref[...].astype(bf16), Wab_ref[...], preferred_element_type=f32)
           + jnp.dot(hd_ref[...].astype(bf16), Wbb_ref[...], preferred_element_type=f32)
           + jnp.dot(ea.astype(bf16), Web_ref[...], preferred_element_type=f32)
           + bf_ref[...])
    preb = pre.astype(bf16)
    act = preb * jax.nn.sigmoid(preb)
    a_node = act[:, :H]
    a_coord = act[:, H:]
    m_ref[...] = jnp.dot(a_node, W2_ref[...], preferred_element_type=f32) + b2_ref[...]
    cw_ref[...] = jnp.sum(a_coord * wc2_ref[...], axis=1)


def _sc_scatter(h, x0, x1, x2, pxs, m, cw, src, dst):
    N, D = h.shape
    E = src.shape[0]
    HC = D // 2
    K = 200
    nh = E // 16 // K
    KX = 320
    ncx = E // 2 // KX
    rows_t = (N // 16) & ~7
    tail0 = rows_t * 16
    ntail = N - tail0
    mesh = plsc.VectorSubcoreMesh(core_axis_name="c", subcore_axis_name="s")

    @functools.partial(
        pl.kernel, mesh=mesh,
        out_type=[jax.ShapeDtypeStruct((N, D), jnp.float32)]
                 + [jax.ShapeDtypeStruct((N,), jnp.float32)] * 6,
        scratch_types=[pltpu.VMEM_SHARED((N, HC), jnp.float32),
                       pltpu.VMEM_SHARED((N,), jnp.float32),
                       pltpu.VMEM_SHARED((N,), jnp.float32),
                       pltpu.VMEM_SHARED((N,), jnp.float32),
                       pltpu.VMEM((K,), jnp.int32),
                       pltpu.VMEM((K, HC), jnp.float32),
                       pltpu.VMEM((KX,), jnp.int32),
                       pltpu.VMEM((KX,), jnp.int32),
                       pltpu.VMEM((KX,), jnp.float32),
                       pltpu.VMEM((KX,), jnp.float32),
                       pltpu.VMEM((KX,), jnp.float32),
                       pltpu.VMEM((KX,), jnp.float32),
                       pltpu.SemaphoreType.DMA],
    )
    def sk(h_hbm, x0_hbm, x1_hbm, x2_hbm,
           i00, i01, i02, i10, i11, i12, m_hbm, cw_hbm,
           src_hbm, dst_hbm, ho_hbm, p00, p01, p02, p10, p11, p12,
           hacc, xa0, xa1, xa2, dstv, mv, srcv, dstxv, cwv, xsv, xdv, cuv,
           sem):
        c = lax.axis_index("c")
        s = lax.axis_index("s")
        r0 = pl.multiple_of(s * rows_t, 8)
        col = pl.multiple_of(c * HC, HC)
        xaccs = [xa0, xa1, xa2]
        xcols = [x0_hbm, x1_hbm, x2_hbm]
        pltpu.sync_copy(h_hbm.at[pl.ds(r0, rows_t), pl.ds(col, HC)],
                        hacc.at[pl.ds(r0, rows_t), :])
        if ntail:
            @pl.when(s == 0)
            def _():
                pltpu.sync_copy(h_hbm.at[pl.ds(tail0, ntail), pl.ds(col, HC)],
                                hacc.at[pl.ds(tail0, ntail), :])
        @pl.when((s == 0) & (c == 0))
        def _():
            for q, src_ref in enumerate([i00, i01, i02]):
                pltpu.sync_copy(src_ref, xaccs[q])
        @pl.when((s == 0) & (c == 1))
        def _():
            for q, src_ref in enumerate([i10, i11, i12]):
                pltpu.sync_copy(src_ref, xaccs[q])
        plsc.subcore_barrier()

        hbase = s * (E // 16)
        def hchunk(j, _):
            off = pl.multiple_of(hbase + j * K, 8)
            pltpu.sync_copy(dst_hbm.at[pl.ds(off, K)], dstv)
            pltpu.sync_copy(m_hbm.at[pl.ds(off, K), pl.ds(col, HC)], mv)
            pltpu.sync_copy(mv, hacc.at[dstv], add=True)
            return _
        lax.fori_loop(0, nh, hchunk, 0)

        nj = (ncx - s + 15) // 16
        def xchunk(j, _):
            cidx = s + j * 16
            off = pl.multiple_of(c * (E // 2) + cidx * KX, 8)
            pltpu.sync_copy(src_hbm.at[pl.ds(off, KX)], srcv)
            pltpu.sync_copy(dst_hbm.at[pl.ds(off, KX)], dstxv)
            pltpu.sync_copy(cw_hbm.at[pl.ds(off, KX)], cwv)
            for q in range(3):
                pltpu.async_copy(xcols[q].at[srcv], xsv, sem).wait()
                pltpu.async_copy(xcols[q].at[dstxv], xdv, sem).wait()
                for g in range(KX // 16):
                    d16 = pl.ds(g * 16, 16)
                    cuv[d16] = cwv[d16] * (xsv[d16] - xdv[d16])
                pltpu.sync_copy(cuv, xaccs[q].at[dstxv], add=True)
            return _
        lax.fori_loop(0, nj, xchunk, 0)
        plsc.subcore_barrier()

        pltpu.sync_copy(hacc.at[pl.ds(r0, rows_t), :],
                        ho_hbm.at[pl.ds(r0, rows_t), pl.ds(col, HC)])
        if ntail:
            @pl.when(s == 0)
            def _():
                pltpu.sync_copy(hacc.at[pl.ds(tail0, ntail), :],
                                ho_hbm.at[pl.ds(tail0, ntail), pl.ds(col, HC)])
        @pl.when((s == 0) & (c == 0))
        def _():
            for q, dst_ref in enumerate([p00, p01, p02]):
                pltpu.sync_copy(xaccs[q], dst_ref)
        @pl.when((s == 0) & (c == 1))
        def _():
            for q, dst_ref in enumerate([p10, p11, p12]):
                pltpu.sync_copy(xaccs[q], dst_ref)

    return sk(h, x0, x1, x2, *pxs, m, cw, src, dst)


def _pick_block(E):
    for b in (256, 128, 64, 32, 16, 8):
        if E % b == 0:
            return b
    return 8


def kernel(h, x, edge_index, edge_dist, W1, b1, W2, b2, Wc1, bc1, Wc2, We1, be1, We2, be2):
    N, D = h.shape
    E = edge_dist.shape[0]
    H = W1.shape[1]
    ED = We2.shape[0]
    B = _pick_block(E)
    src = edge_index[0]
    dst = edge_index[1]

    bf16 = jnp.bfloat16
    Wab = jnp.concatenate([W1[:D], Wc1[:D]], axis=1).astype(bf16)
    Wbb = jnp.concatenate([W1[D:2 * D], Wc1[D:2 * D]], axis=1).astype(bf16)
    Web = jnp.concatenate([W1[2 * D:], Wc1[2 * D:]], axis=1).astype(bf16)
    bf = jnp.concatenate([b1, bc1])[None, :]
    b2r = b2[None, :]
    wc2r = Wc2[:, 0][None, :]
    be1r = be1[None, :]
    be2r = be2[None, :]

    weights = (Wab, Wbb, Web, bf, W2.astype(bf16), b2r, wc2r,
               We1, be1r, We2, be2r)
    full = lambda r, c: pl.BlockSpec((r, c), lambda i: (0, 0))

    def edge_mlp(hs, hd, ed2):
        ES = hs.shape[0]
        return pl.pallas_call(
            functools.partial(_edge_mlp_body, H=H),
            grid=(ES // B,),
            in_specs=[
                pl.BlockSpec((B, D), lambda i: (i, 0)),
                pl.BlockSpec((B, D), lambda i: (i, 0)),
                pl.BlockSpec((B, 1), lambda i: (i, 0)),
                full(D, 2 * H), full(D, 2 * H), full(ED, 2 * H), full(1, 2 * H),
                full(H, D), full(1, D), full(1, H),
                full(1, ED), full(1, ED), full(ED, ED), full(1, ED),
            ],
            out_specs=[
                pl.BlockSpec((B, D), lambda i: (i, 0)),
                pl.BlockSpec((B,), lambda i: (i,)),
            ],
            out_shape=[
                jax.ShapeDtypeStruct((ES, D), jnp.float32),
                jax.ShapeDtypeStruct((ES,), jnp.float32),
            ],
        )(hs, hd, ed2, *weights)

    S = 5 if E % (5 * 32 * 200) == 0 else 1
    ES = E // S
    ho = h
    pxs = [jnp.zeros((N,), jnp.float32)] * 6
    x0, x1, x2 = x[:, 0], x[:, 1], x[:, 2]
    for si in range(S):
        sl = slice(si * ES, (si + 1) * ES)
        hs, hd = _sc_gather_pair(h, src[sl], dst[sl])
        m, cw = edge_mlp(hs, hd, edge_dist[sl, None])
        ho, *pxs = _sc_scatter(ho, x0, x1, x2, pxs, m, cw, src[sl], dst[sl])
    p00, p01, p02, p10, p11, p12 = pxs
    x_out = x + jnp.stack([p00 + p10, p01 + p11, p02 + p12], axis=1)
    return (ho, x_out)
